# Initial kernel scaffold; baseline (speedup 1.0000x reference)
import jax
import jax.numpy as jnp
from jax.experimental import pallas as pl


def kernel(hidden_states, router_logits, W1, W2):
    raise NotImplementedError("write your pallas kernel here")


# speedup vs baseline: 2.9343x; 2.9343x over previous
import functools

import jax
import jax.numpy as jnp
from jax import lax
from jax.experimental import pallas as pl
from jax.experimental.pallas import tpu as pltpu

T = 2048
D = 2048
FF = 2048
E = 8
TOPK = 2
M = T * TOPK

BM = 256
M_BLOCKS = M // BM
NUM_ITEMS = M_BLOCKS + E - 1
BF = 1024
N_FF = FF // BF


def _routing_metadata(flat_experts, sorted_idx):
    counts = jnp.bincount(flat_experts, length=E)
    offsets = jnp.concatenate([jnp.zeros((1,), jnp.int32),
                               jnp.cumsum(counts).astype(jnp.int32)])
    b_grid = jnp.arange(M_BLOCKS, dtype=jnp.int32)[:, None]
    e_grid = jnp.arange(E, dtype=jnp.int32)[None, :]
    lo_g = jnp.maximum(offsets[:-1][None, :], b_grid * BM)
    hi_g = jnp.minimum(offsets[1:][None, :], (b_grid + 1) * BM)
    valid = lo_g < hi_g
    key = jnp.where(valid, b_grid * E + e_grid, 1 << 30).reshape(-1)
    order = jnp.argsort(key)[:NUM_ITEMS]
    kf = key[order]
    pad = kf >= (1 << 30)
    b_arr = jnp.where(pad, M_BLOCKS - 1, kf // E).astype(jnp.int32)
    e_arr = jnp.where(pad, E - 1, kf % E).astype(jnp.int32)
    lo_arr = jnp.where(pad, 0, lo_g.reshape(-1)[order] - b_arr * BM).astype(jnp.int32)
    hi_arr = jnp.where(pad, 0, hi_g.reshape(-1)[order] - b_arr * BM).astype(jnp.int32)
    first = jnp.concatenate([jnp.ones((1,), jnp.int32),
                             (b_arr[1:] != b_arr[:-1]).astype(jnp.int32)])
    return b_arr, e_arr, lo_arr, hi_arr, first


def _fc1_body(b_ref, e_ref, lo_ref, hi_ref, first_ref, x_ref, w1a_ref, w1b_ref,
              h_ref):
    i = pl.program_id(1)
    lo = lo_ref[i]
    hi = hi_ref[i]
    first = first_ref[i]

    @pl.when(hi > lo)
    def _():
        x = x_ref[...]
        p = jnp.dot(x, w1a_ref[0], preferred_element_type=jnp.float32)
        g = jnp.dot(x, w1b_ref[0], preferred_element_type=jnp.float32)
        val = jax.nn.silu(p) * g
        rows = lax.broadcasted_iota(jnp.int32, (BM, BF), 0)
        val = jnp.where((rows >= lo) & (rows < hi), val, 0.0)

        @pl.when(first == 1)
        def _():
            h_ref[...] = val

        @pl.when(first == 0)
        def _():
            h_ref[...] += val


def _fc2_body(b_ref, e_ref, lo_ref, hi_ref, first_ref, h_ref, w2_ref, s_ref,
              y_ref):
    i = pl.program_id(0)
    lo = lo_ref[i]
    hi = hi_ref[i]
    first = first_ref[i]

    @pl.when(hi > lo)
    def _():
        y = jnp.dot(h_ref[...], w2_ref[0], preferred_element_type=jnp.float32)
        y = y * s_ref[...]
        rows = lax.broadcasted_iota(jnp.int32, (BM, D), 0)
        val = jnp.where((rows >= lo) & (rows < hi), y, 0.0)

        @pl.when(first == 1)
        def _():
            y_ref[...] = val

        @pl.when(first == 0)
        def _():
            y_ref[...] += val


def _grouped_mlp(meta, xs, W1, W2, s_sorted, interpret=False):
    b_arr, e_arr, lo_arr, hi_arr, first = meta
    fc1 = pl.pallas_call(
        _fc1_body,
        grid_spec=pltpu.PrefetchScalarGridSpec(
            num_scalar_prefetch=5,
            grid=(N_FF, NUM_ITEMS),
            in_specs=[
                pl.BlockSpec((BM, D), lambda j, i, b, e, lo, hi, fs: (b[i], 0)),
                pl.BlockSpec((1, D, BF),
                             lambda j, i, b, e, lo, hi, fs: (e[i], 0, j)),
                pl.BlockSpec((1, D, BF),
                             lambda j, i, b, e, lo, hi, fs: (e[i], 0, N_FF + j)),
            ],
            out_specs=pl.BlockSpec((BM, BF),
                                   lambda j, i, b, e, lo, hi, fs: (b[i], j)),
        ),
        out_shape=jax.ShapeDtypeStruct((M, FF), jnp.float32),
        interpret=interpret,
    )
    h = fc1(b_arr, e_arr, lo_arr, hi_arr, first, xs, W1, W1)
    fc2 = pl.pallas_call(
        _fc2_body,
        grid_spec=pltpu.PrefetchScalarGridSpec(
            num_scalar_prefetch=5,
            grid=(NUM_ITEMS,),
            in_specs=[
                pl.BlockSpec((BM, FF), lambda i, b, e, lo, hi, fs: (b[i], 0)),
                pl.BlockSpec((1, FF, D), lambda i, b, e, lo, hi, fs: (e[i], 0, 0)),
                pl.BlockSpec((BM, 1), lambda i, b, e, lo, hi, fs: (b[i], 0)),
            ],
            out_specs=pl.BlockSpec((BM, D),
                                   lambda i, b, e, lo, hi, fs: (b[i], 0)),
        ),
        out_shape=jax.ShapeDtypeStruct((M, D), jnp.float32),
        interpret=interpret,
    )
    return fc2(b_arr, e_arr, lo_arr, hi_arr, first, h, W2, s_sorted)


def kernel(hidden_states, router_logits, W1, W2):
    top_logits, top_indices = lax.top_k(router_logits, TOPK)
    scores = jax.nn.softmax(top_logits, axis=-1)
    flat = top_indices.reshape(-1).astype(jnp.int32)
    sorted_idx = jnp.argsort(flat, stable=True).astype(jnp.int32)
    meta = _routing_metadata(flat, sorted_idx)

    xs = hidden_states[sorted_idx // TOPK]
    s_sorted = scores.reshape(-1)[sorted_idx][:, None]

    ys = _grouped_mlp(meta, xs, W1, W2, s_sorted)

    unperm = jnp.zeros((M, D), jnp.float32).at[sorted_idx].set(ys)
    return unperm.reshape(T, TOPK, D).sum(axis=1)
